# baseline (speedup 1.0000x reference)
import functools

import jax
import jax.numpy as jnp
from jax import lax
from jax.experimental import pallas as pl
from jax.experimental.pallas import tpu as pltpu
from jax.experimental.pallas import tpu_sc as plsc

N = 10000
NPAD = 10240
E = 320000
DIN = 128
DH = 64
HALF = 32
K = 8

NSC = 2
NTILE = 16
WIN = 512
NWIN = 40
ETILE = WIN * NWIN
EPAD = NTILE * ETILE
RT = NPAD // NTILE
CH = 160
NCH = RT // CH
BN = 1024

_mesh = plsc.VectorSubcoreMesh(core_axis_name="c", subcore_axis_name="s")

_sc_params = pltpu.CompilerParams(
    needs_layout_passes=False, use_tc_tiling_on_sc=False)


def _splat(ref, idx):
    return plsc.load_gather(ref, [jnp.full((16,), idx, jnp.int32)])


def _mm_body(x_ref, w_ref, o_ref):
    acc = lax.dot_general(
        x_ref[...], w_ref[...],
        (((1,), (0,)), ((), ())),
        precision=lax.Precision.HIGHEST,
        preferred_element_type=jnp.float32,
    )
    for cid in range(NSC):
        for k in range(K):
            g = cid * K + k
            o_ref[cid, k] = acc[:, g * HALF:(g + 1) * HALF]


def _projections(x_pad, W2):
    return pl.pallas_call(
        _mm_body,
        grid=(NPAD // BN,),
        in_specs=[
            pl.BlockSpec((BN, DIN), lambda nb: (nb, 0)),
            pl.BlockSpec((DIN, NSC * K * HALF), lambda nb: (0, 0)),
        ],
        out_specs=pl.BlockSpec((NSC, K, BN, HALF), lambda nb: (0, 0, nb, 0)),
        out_shape=jax.ShapeDtypeStruct((NSC, K, NPAD, HALF), jnp.float32),
    )(x_pad, W2)


@functools.partial(
    pl.kernel,
    out_type=jax.ShapeDtypeStruct((NSC, NPAD), jnp.float32),
    mesh=_mesh,
    compiler_params=_sc_params,
    scratch_types=[
        pltpu.VMEM((ETILE,), jnp.int32),
        pltpu.VMEM((WIN,), jnp.float32),
        pltpu.VMEM((RT,), jnp.float32),
        pltpu.VMEM_SHARED((NPAD,), jnp.float32),
    ],
)
def _deg_kernel(rowi_hbm, dis_hbm, rowi_v, ones_v, degs_v, deg_sh):
    tid = lax.axis_index("s")
    cid = lax.axis_index("c")
    base = tid * RT

    pltpu.sync_copy(rowi_hbm.at[tid], rowi_v)

    @pl.loop(0, WIN, step=16)
    def _(j):
        ones_v[pl.ds(j, 16)] = jnp.full((16,), 1.0, jnp.float32)

    @pl.loop(0, RT, step=16)
    def _(j):
        degs_v[pl.ds(j, 16)] = jnp.zeros((16,), jnp.float32)

    pltpu.sync_copy(degs_v, deg_sh.at[pl.ds(base, RT)])
    plsc.subcore_barrier()

    @pl.loop(0, NWIN)
    def _(w):
        pltpu.sync_copy(ones_v, deg_sh.at[rowi_v.at[pl.ds(w * WIN, WIN)]], add=True)

    plsc.subcore_barrier()
    pltpu.sync_copy(deg_sh.at[pl.ds(base, RT)], degs_v)

    @pl.loop(0, RT, step=16)
    def _(j):
        d = degs_v[pl.ds(j, 16)]
        bits = lax.bitcast_convert_type(d, jnp.int32)
        y = lax.bitcast_convert_type(0x5F3759DF - (bits >> 1), jnp.float32)
        y = y * (1.5 - 0.5 * d * y * y)
        y = y * (1.5 - 0.5 * d * y * y)
        y = y * (1.5 - 0.5 * d * y * y)
        y = y * (1.5 - 0.5 * d * y * y)
        degs_v[pl.ds(j, 16)] = jnp.where(d > 0.0, y, 0.0)

    pltpu.sync_copy(degs_v, dis_hbm.at[cid, pl.ds(base, RT)])


@functools.partial(
    pl.kernel,
    out_type=[
        jax.ShapeDtypeStruct((NSC, NPAD, HALF), jnp.float32),
        jax.ShapeDtypeStruct((NSC * NPAD, HALF), jnp.float32),
        jax.ShapeDtypeStruct((NSC, NPAD, HALF), jnp.float32),
        jax.ShapeDtypeStruct((NSC, NPAD, HALF), jnp.float32),
    ],
    mesh=_mesh,
    compiler_params=_sc_params,
    scratch_types=[
        pltpu.VMEM((ETILE,), jnp.int32),
        pltpu.VMEM((ETILE,), jnp.int32),
        pltpu.VMEM((WIN, HALF), jnp.float32),
        pltpu.VMEM((WIN, HALF), jnp.float32),
        pltpu.SemaphoreType.DMA,
        pltpu.SemaphoreType.DMA,
        pltpu.SemaphoreType.DMA,
        pltpu.SemaphoreType.DMA,
        pltpu.VMEM((RT,), jnp.float32),
        pltpu.VMEM((RT,), jnp.float32),
        pltpu.VMEM((CH, HALF), jnp.float32),
        pltpu.VMEM((CH, HALF), jnp.float32),
        pltpu.VMEM((CH, HALF), jnp.float32),
        pltpu.VMEM((CH, HALF), jnp.float32),
        pltpu.VMEM((CH, HALF), jnp.float32),
        pltpu.VMEM((CH, HALF), jnp.float32),
        pltpu.VMEM((HALF,), jnp.float32),
        pltpu.VMEM_SHARED((NPAD, HALF), jnp.float32),
    ],
)
def _cheb_kernel(rowi_hbm, coli_hbm, dis_hbm, c_hbm, bias_hbm,
                 out_hbm, htab_hbm, b0_hbm, b1_hbm,
                 coli_v, rowi_v, gbuf0, gbuf1, gsem0, gsem1, ssem0, ssem1,
                 dis_v, dis2_v, zero_v,
                 accv, cv, bpv, bnv, hv, biasv,
                 acc_sh):
    tid = lax.axis_index("s")
    cid = lax.axis_index("c")
    base = tid * RT
    hoff = cid * NPAD

    pltpu.sync_copy(coli_hbm.at[tid], coli_v)
    pltpu.sync_copy(rowi_hbm.at[tid], rowi_v)
    pltpu.sync_copy(dis_hbm.at[cid, pl.ds(base, RT)], dis_v)
    pltpu.sync_copy(bias_hbm.at[cid], biasv)

    @pl.loop(0, ETILE, step=16)
    def _(j):
        coli_v[pl.ds(j, 16)] = coli_v[pl.ds(j, 16)] + hoff

    @pl.loop(0, RT, step=16)
    def _(j):
        dis2_v[pl.ds(j, 16)] = dis_v[pl.ds(j, 16)] * 2.0

    @pl.loop(0, CH)
    def _(i):
        zero_v[i, pl.ds(0, 16)] = jnp.zeros((16,), jnp.float32)
        zero_v[i, pl.ds(16, 16)] = jnp.zeros((16,), jnp.float32)

    @pl.loop(0, NCH)
    def _(ch):
        r0 = base + ch * CH
        pltpu.sync_copy(c_hbm.at[cid, K - 1, pl.ds(r0, CH)], cv)

        @pl.loop(0, CH)
        def _(i):
            s = _splat(dis_v, ch * CH + i)
            hv[i, pl.ds(0, 16)] = cv[i, pl.ds(0, 16)] * s
            hv[i, pl.ds(16, 16)] = cv[i, pl.ds(16, 16)] * s

        pltpu.sync_copy(cv, b0_hbm.at[cid, pl.ds(r0, CH)])
        pltpu.sync_copy(hv, htab_hbm.at[pl.ds(hoff + r0, CH)])
        pltpu.sync_copy(zero_v, acc_sh.at[pl.ds(r0, CH)])

    plsc.subcore_barrier()

    for j, k in enumerate(range(K - 2, -1, -1)):
        first = j == 0
        final = k == 0
        bbuf = b1_hbm if j % 2 == 0 else b0_hbm

        pltpu.async_copy(
            htab_hbm.at[coli_v.at[pl.ds(0, WIN)]], gbuf0, gsem0)
        pltpu.async_copy(
            htab_hbm.at[coli_v.at[pl.ds(WIN, WIN)]], gbuf1, gsem1)

        @pl.loop(0, NWIN, step=2)
        def _(w):
            pltpu.make_async_copy(
                htab_hbm.at[coli_v.at[pl.ds(0, WIN)]], gbuf0, gsem0).wait()
            pltpu.async_copy(
                gbuf0, acc_sh.at[rowi_v.at[pl.ds(w * WIN, WIN)]], ssem0, add=True)
            pltpu.make_async_copy(
                htab_hbm.at[coli_v.at[pl.ds(0, WIN)]], gbuf1, gsem1).wait()
            pltpu.async_copy(
                gbuf1, acc_sh.at[rowi_v.at[pl.ds((w + 1) * WIN, WIN)]], ssem1, add=True)

            pltpu.make_async_copy(
                gbuf0, acc_sh.at[rowi_v.at[pl.ds(0, WIN)]], ssem0).wait()

            @pl.when(w + 2 < NWIN)
            def _():
                pltpu.async_copy(
                    htab_hbm.at[coli_v.at[pl.ds((w + 2) * WIN, WIN)]], gbuf0, gsem0)

            pltpu.make_async_copy(
                gbuf1, acc_sh.at[rowi_v.at[pl.ds(0, WIN)]], ssem1).wait()

            @pl.when(w + 3 < NWIN)
            def _():
                pltpu.async_copy(
                    htab_hbm.at[coli_v.at[pl.ds((w + 3) * WIN, WIN)]], gbuf1, gsem1)

        plsc.subcore_barrier()

        @pl.loop(0, NCH)
        def _(ch):
            r0 = base + ch * CH
            pltpu.sync_copy(acc_sh.at[pl.ds(r0, CH)], accv)
            pltpu.sync_copy(c_hbm.at[cid, k, pl.ds(r0, CH)], cv)
            if not first:
                pltpu.sync_copy(bbuf.at[cid, pl.ds(r0, CH)], bpv)

            @pl.loop(0, CH)
            def _(i):
                s2 = _splat(dis_v if final else dis2_v, ch * CH + i)
                if not final:
                    s1 = _splat(dis_v, ch * CH + i)
                for f in (0, 16):
                    t = cv[i, pl.ds(f, 16)] - s2 * accv[i, pl.ds(f, 16)]
                    if not first:
                        t = t - bpv[i, pl.ds(f, 16)]
                    if final:
                        t = t + biasv[pl.ds(f, 16)]
                        t = jnp.where(t > 0.0, t, jnp.exp(t) - 1.0)
                        bnv[i, pl.ds(f, 16)] = t
                    else:
                        bnv[i, pl.ds(f, 16)] = t
                        hv[i, pl.ds(f, 16)] = t * s1

            if final:
                pltpu.sync_copy(bnv, out_hbm.at[cid, pl.ds(r0, CH)])
            else:
                pltpu.sync_copy(bnv, bbuf.at[cid, pl.ds(r0, CH)])
                pltpu.sync_copy(hv, htab_hbm.at[pl.ds(hoff + r0, CH)])
                pltpu.sync_copy(zero_v, acc_sh.at[pl.ds(r0, CH)])

        if not final:
            plsc.subcore_barrier()


def kernel(x, edge_index, W, b):
    pad_tgt = N + (jnp.arange(EPAD - E, dtype=jnp.int32) % (NPAD - N))
    row = jnp.concatenate([edge_index[0].astype(jnp.int32), pad_tgt])
    col = jnp.concatenate([edge_index[1].astype(jnp.int32), pad_tgt])
    row = row.reshape(NTILE, ETILE)
    col = col.reshape(NTILE, ETILE)
    x_pad = jnp.pad(x, ((0, NPAD - N), (0, 0)))
    bias = b.reshape(NSC, HALF)

    W2 = W.reshape(K, DIN, NSC, HALF).transpose(1, 2, 0, 3).reshape(DIN, NSC * K * HALF)
    c = _projections(x_pad, W2)
    dis = _deg_kernel(row)
    out_halves, _, _, _ = _cheb_kernel(row, col, dis, c, bias)
    return out_halves.transpose(1, 0, 2).reshape(NPAD, DH)[:N]


# speedup vs baseline: 1.3329x; 1.3329x over previous
import functools

import jax
import jax.numpy as jnp
from jax import lax
from jax.experimental import pallas as pl
from jax.experimental.pallas import tpu as pltpu
from jax.experimental.pallas import tpu_sc as plsc

N = 10000
NPAD = 10240
E = 320000
DIN = 128
DH = 64
HALF = 32
K = 8

NSC = 2
NTILE = 16
WIN = 512
NWIN = 40
ETILE = WIN * NWIN
EPAD = NTILE * ETILE
RT = NPAD // NTILE
CH = 160
NCH = RT // CH
BN = 1024

_mesh = plsc.VectorSubcoreMesh(core_axis_name="c", subcore_axis_name="s")

_sc_params = pltpu.CompilerParams(
    needs_layout_passes=False, use_tc_tiling_on_sc=False)


def _splat(ref, idx):
    return plsc.load_gather(ref, [jnp.full((16,), idx, jnp.int32)])


def _mm_body(x_ref, w_ref, o_ref):
    acc = lax.dot_general(
        x_ref[...], w_ref[...],
        (((1,), (0,)), ((), ())),
        precision=lax.Precision.HIGHEST,
        preferred_element_type=jnp.float32,
    )
    zpad = jnp.zeros((acc.shape[0], DIN - HALF), jnp.float32)
    for cid in range(NSC):
        for k in range(K):
            g = cid * K + k
            o_ref[cid, k] = jnp.concatenate(
                [acc[:, g * HALF:(g + 1) * HALF], zpad], axis=1)


def _projections(x_pad, W2):
    return pl.pallas_call(
        _mm_body,
        grid=(NPAD // BN,),
        in_specs=[
            pl.BlockSpec((BN, DIN), lambda nb: (nb, 0)),
            pl.BlockSpec((DIN, NSC * K * HALF), lambda nb: (0, 0)),
        ],
        out_specs=pl.BlockSpec((NSC, K, BN, DIN), lambda nb: (0, 0, nb, 0)),
        out_shape=jax.ShapeDtypeStruct((NSC, K, NPAD, DIN), jnp.float32),
    )(x_pad, W2)


@functools.partial(
    pl.kernel,
    out_type=jax.ShapeDtypeStruct((NSC, NPAD), jnp.float32),
    mesh=_mesh,
    compiler_params=_sc_params,
    scratch_types=[
        pltpu.VMEM((ETILE,), jnp.int32),
        pltpu.VMEM((WIN,), jnp.float32),
        pltpu.VMEM((RT,), jnp.float32),
        pltpu.VMEM_SHARED((NPAD,), jnp.float32),
    ],
)
def _deg_kernel(rowi_hbm, dis_hbm, rowi_v, ones_v, degs_v, deg_sh):
    tid = lax.axis_index("s")
    cid = lax.axis_index("c")
    base = tid * RT

    pltpu.sync_copy(rowi_hbm.at[tid], rowi_v)

    @pl.loop(0, WIN, step=16)
    def _(j):
        ones_v[pl.ds(j, 16)] = jnp.full((16,), 1.0, jnp.float32)

    @pl.loop(0, RT, step=16)
    def _(j):
        degs_v[pl.ds(j, 16)] = jnp.zeros((16,), jnp.float32)

    pltpu.sync_copy(degs_v, deg_sh.at[pl.ds(base, RT)])
    plsc.subcore_barrier()

    @pl.loop(0, NWIN)
    def _(w):
        pltpu.sync_copy(ones_v, deg_sh.at[rowi_v.at[pl.ds(w * WIN, WIN)]], add=True)

    plsc.subcore_barrier()
    pltpu.sync_copy(deg_sh.at[pl.ds(base, RT)], degs_v)

    @pl.loop(0, RT, step=16)
    def _(j):
        d = degs_v[pl.ds(j, 16)]
        bits = lax.bitcast_convert_type(d, jnp.int32)
        y = lax.bitcast_convert_type(0x5F3759DF - (bits >> 1), jnp.float32)
        y = y * (1.5 - 0.5 * d * y * y)
        y = y * (1.5 - 0.5 * d * y * y)
        y = y * (1.5 - 0.5 * d * y * y)
        y = y * (1.5 - 0.5 * d * y * y)
        degs_v[pl.ds(j, 16)] = jnp.where(d > 0.0, y, 0.0)

    pltpu.sync_copy(degs_v, dis_hbm.at[cid, pl.ds(base, RT)])


@functools.partial(
    pl.kernel,
    out_type=[
        jax.ShapeDtypeStruct((NSC, NPAD, HALF), jnp.float32),
        jax.ShapeDtypeStruct((NSC * NPAD, HALF), jnp.float32),
        jax.ShapeDtypeStruct((NSC, NPAD, HALF), jnp.float32),
        jax.ShapeDtypeStruct((NSC, NPAD, HALF), jnp.float32),
    ],
    mesh=_mesh,
    compiler_params=_sc_params,
    scratch_types=[
        pltpu.VMEM((ETILE,), jnp.int32),
        pltpu.VMEM((ETILE,), jnp.int32),
        pltpu.VMEM((WIN, HALF), jnp.float32),
        pltpu.VMEM((WIN, HALF), jnp.float32),
        pltpu.SemaphoreType.DMA,
        pltpu.SemaphoreType.DMA,
        pltpu.SemaphoreType.DMA,
        pltpu.SemaphoreType.DMA,
        pltpu.VMEM((RT,), jnp.float32),
        pltpu.VMEM((RT,), jnp.float32),
        pltpu.VMEM((CH, HALF), jnp.float32),
        pltpu.VMEM((CH, HALF), jnp.float32),
        pltpu.VMEM((CH, HALF), jnp.float32),
        pltpu.VMEM((CH, HALF), jnp.float32),
        pltpu.VMEM((CH, HALF), jnp.float32),
        pltpu.VMEM((CH, HALF), jnp.float32),
        pltpu.VMEM((HALF,), jnp.float32),
        pltpu.VMEM_SHARED((NPAD, HALF), jnp.float32),
    ],
)
def _cheb_kernel(rowi_hbm, coli_hbm, dis_hbm, c_hbm, bias_hbm,
                 out_hbm, htab_hbm, b0_hbm, b1_hbm,
                 coli_v, rowi_v, gbuf0, gbuf1, gsem0, gsem1, ssem0, ssem1,
                 dis_v, dis2_v, zero_v,
                 accv, cv, bpv, bnv, hv, biasv,
                 acc_sh):
    tid = lax.axis_index("s")
    cid = lax.axis_index("c")
    base = tid * RT
    hoff = cid * NPAD

    pltpu.sync_copy(coli_hbm.at[tid], coli_v)
    pltpu.sync_copy(rowi_hbm.at[tid], rowi_v)
    pltpu.sync_copy(dis_hbm.at[cid, pl.ds(base, RT)], dis_v)
    pltpu.sync_copy(bias_hbm.at[cid], biasv)

    @pl.loop(0, ETILE, step=16)
    def _(j):
        coli_v[pl.ds(j, 16)] = coli_v[pl.ds(j, 16)] + hoff

    @pl.loop(0, RT, step=16)
    def _(j):
        dis2_v[pl.ds(j, 16)] = dis_v[pl.ds(j, 16)] * 2.0

    @pl.loop(0, CH)
    def _(i):
        zero_v[i, pl.ds(0, 16)] = jnp.zeros((16,), jnp.float32)
        zero_v[i, pl.ds(16, 16)] = jnp.zeros((16,), jnp.float32)

    @pl.loop(0, NCH)
    def _(ch):
        r0 = base + ch * CH
        pltpu.sync_copy(c_hbm.at[cid, K - 1, pl.ds(r0, CH), pl.ds(0, HALF)], cv)

        @pl.loop(0, CH)
        def _(i):
            s = _splat(dis_v, ch * CH + i)
            hv[i, pl.ds(0, 16)] = cv[i, pl.ds(0, 16)] * s
            hv[i, pl.ds(16, 16)] = cv[i, pl.ds(16, 16)] * s

        pltpu.sync_copy(cv, b0_hbm.at[cid, pl.ds(r0, CH)])
        pltpu.sync_copy(hv, htab_hbm.at[pl.ds(hoff + r0, CH)])
        pltpu.sync_copy(zero_v, acc_sh.at[pl.ds(r0, CH)])

    plsc.subcore_barrier()

    for j, k in enumerate(range(K - 2, -1, -1)):
        first = j == 0
        final = k == 0
        bbuf = b1_hbm if j % 2 == 0 else b0_hbm

        pltpu.async_copy(
            htab_hbm.at[coli_v.at[pl.ds(0, WIN)]], gbuf0, gsem0)

        @pl.loop(0, NWIN, step=2)
        def _(w):
            pltpu.async_copy(
                htab_hbm.at[coli_v.at[pl.ds((w + 1) * WIN, WIN)]], gbuf1, gsem1)
            pltpu.make_async_copy(
                htab_hbm.at[coli_v.at[pl.ds(0, WIN)]], gbuf0, gsem0).wait()
            pltpu.sync_copy(gbuf0, acc_sh.at[rowi_v.at[pl.ds(w * WIN, WIN)]], add=True)

            @pl.when(w + 2 < NWIN)
            def _():
                pltpu.async_copy(
                    htab_hbm.at[coli_v.at[pl.ds((w + 2) * WIN, WIN)]], gbuf0, gsem0)

            pltpu.make_async_copy(
                htab_hbm.at[coli_v.at[pl.ds(0, WIN)]], gbuf1, gsem1).wait()
            pltpu.sync_copy(gbuf1, acc_sh.at[rowi_v.at[pl.ds((w + 1) * WIN, WIN)]], add=True)

        plsc.subcore_barrier()

        @pl.loop(0, NCH)
        def _(ch):
            r0 = base + ch * CH
            pltpu.sync_copy(acc_sh.at[pl.ds(r0, CH)], accv)
            pltpu.sync_copy(c_hbm.at[cid, k, pl.ds(r0, CH), pl.ds(0, HALF)], cv)
            if not first:
                pltpu.sync_copy(bbuf.at[cid, pl.ds(r0, CH)], bpv)

            @pl.loop(0, CH)
            def _(i):
                s2 = _splat(dis_v if final else dis2_v, ch * CH + i)
                if not final:
                    s1 = _splat(dis_v, ch * CH + i)
                for f in (0, 16):
                    t = cv[i, pl.ds(f, 16)] - s2 * accv[i, pl.ds(f, 16)]
                    if not first:
                        t = t - bpv[i, pl.ds(f, 16)]
                    if final:
                        t = t + biasv[pl.ds(f, 16)]
                        t = jnp.where(t > 0.0, t, jnp.exp(t) - 1.0)
                        bnv[i, pl.ds(f, 16)] = t
                    else:
                        bnv[i, pl.ds(f, 16)] = t
                        hv[i, pl.ds(f, 16)] = t * s1

            if final:
                pltpu.sync_copy(bnv, out_hbm.at[cid, pl.ds(r0, CH)])
            else:
                pltpu.sync_copy(bnv, bbuf.at[cid, pl.ds(r0, CH)])
                pltpu.sync_copy(hv, htab_hbm.at[pl.ds(hoff + r0, CH)])
                pltpu.sync_copy(zero_v, acc_sh.at[pl.ds(r0, CH)])

        if not final:
            plsc.subcore_barrier()


def kernel(x, edge_index, W, b):
    pad_tgt = N + (jnp.arange(EPAD - E, dtype=jnp.int32) % (NPAD - N))
    row = jnp.concatenate([edge_index[0].astype(jnp.int32), pad_tgt])
    col = jnp.concatenate([edge_index[1].astype(jnp.int32), pad_tgt])
    row = row.reshape(NTILE, ETILE)
    col = col.reshape(NTILE, ETILE)
    x_pad = jnp.pad(x, ((0, NPAD - N), (0, 0)))
    bias = b.reshape(NSC, HALF)

    W2 = W.reshape(K, DIN, NSC, HALF).transpose(1, 2, 0, 3).reshape(DIN, NSC * K * HALF)
    c = _projections(x_pad, W2)
    dis = _deg_kernel(row)
    out_halves, _, _, _ = _cheb_kernel(row, col, dis, c, bias)
    return out_halves.transpose(1, 0, 2).reshape(NPAD, DH)[:N]


# speedup vs baseline: 1.4560x; 1.0923x over previous
import functools

import jax
import jax.numpy as jnp
from jax import lax
from jax.experimental import pallas as pl
from jax.experimental.pallas import tpu as pltpu
from jax.experimental.pallas import tpu_sc as plsc

N = 10000
NPAD = 10240
E = 320000
DIN = 128
DH = 64
HALF = 32
K = 8

NSC = 2
NTILE = 16
WIN = 512
NWIN = 40
ETILE = WIN * NWIN
EPAD = NTILE * ETILE
RT = NPAD // NTILE
CH = 160
NCH = RT // CH
BN = 1024

_mesh = plsc.VectorSubcoreMesh(core_axis_name="c", subcore_axis_name="s")

_sc_params = pltpu.CompilerParams(
    needs_layout_passes=False, use_tc_tiling_on_sc=False)


def _splat(ref, idx):
    return plsc.load_gather(ref, [jnp.full((16,), idx, jnp.int32)])


def _mm_body(x_ref, w_ref, o_ref):
    acc = lax.dot_general(
        x_ref[...], w_ref[...],
        (((1,), (0,)), ((), ())),
        precision=lax.Precision.HIGHEST,
        preferred_element_type=jnp.float32,
    )
    for cid in range(NSC):
        for k in range(K):
            g = cid * K + k
            o_ref[cid, k, :, pl.ds(0, HALF)] = acc[:, g * HALF:(g + 1) * HALF]


def _projections(x_pad, W2):
    return pl.pallas_call(
        _mm_body,
        grid=(NPAD // BN,),
        in_specs=[
            pl.BlockSpec((BN, DIN), lambda nb: (nb, 0)),
            pl.BlockSpec((DIN, NSC * K * HALF), lambda nb: (0, 0)),
        ],
        out_specs=pl.BlockSpec((NSC, K, BN, DIN), lambda nb: (0, 0, nb, 0)),
        out_shape=jax.ShapeDtypeStruct((NSC, K, NPAD, DIN), jnp.float32),
    )(x_pad, W2)


@functools.partial(
    pl.kernel,
    out_type=jax.ShapeDtypeStruct((NSC, NPAD), jnp.float32),
    mesh=_mesh,
    compiler_params=_sc_params,
    scratch_types=[
        pltpu.VMEM((ETILE,), jnp.int32),
        pltpu.VMEM((WIN,), jnp.float32),
        pltpu.VMEM((RT,), jnp.float32),
        pltpu.VMEM_SHARED((NPAD,), jnp.float32),
    ],
)
def _deg_kernel(rowi_hbm, dis_hbm, rowi_v, ones_v, degs_v, deg_sh):
    tid = lax.axis_index("s")
    cid = lax.axis_index("c")
    base = tid * RT

    pltpu.sync_copy(rowi_hbm.at[tid], rowi_v)

    @pl.loop(0, WIN, step=16)
    def _(j):
        ones_v[pl.ds(j, 16)] = jnp.full((16,), 1.0, jnp.float32)

    @pl.loop(0, RT, step=16)
    def _(j):
        degs_v[pl.ds(j, 16)] = jnp.zeros((16,), jnp.float32)

    pltpu.sync_copy(degs_v, deg_sh.at[pl.ds(base, RT)])
    plsc.subcore_barrier()

    @pl.loop(0, NWIN)
    def _(w):
        pltpu.sync_copy(ones_v, deg_sh.at[rowi_v.at[pl.ds(w * WIN, WIN)]], add=True)

    plsc.subcore_barrier()
    pltpu.sync_copy(deg_sh.at[pl.ds(base, RT)], degs_v)

    @pl.loop(0, RT, step=16)
    def _(j):
        d = degs_v[pl.ds(j, 16)]
        bits = lax.bitcast_convert_type(d, jnp.int32)
        y = lax.bitcast_convert_type(0x5F3759DF - (bits >> 1), jnp.float32)
        y = y * (1.5 - 0.5 * d * y * y)
        y = y * (1.5 - 0.5 * d * y * y)
        y = y * (1.5 - 0.5 * d * y * y)
        y = y * (1.5 - 0.5 * d * y * y)
        degs_v[pl.ds(j, 16)] = jnp.where(d > 0.0, y, 0.0)

    pltpu.sync_copy(degs_v, dis_hbm.at[cid, pl.ds(base, RT)])


@functools.partial(
    pl.kernel,
    out_type=[
        jax.ShapeDtypeStruct((NPAD, DH), jnp.float32),
        jax.ShapeDtypeStruct((NSC * NPAD, HALF), jnp.float32),
        jax.ShapeDtypeStruct((NSC, NPAD, HALF), jnp.float32),
        jax.ShapeDtypeStruct((NSC, NPAD, HALF), jnp.float32),
    ],
    mesh=_mesh,
    compiler_params=_sc_params,
    scratch_types=[
        pltpu.VMEM((ETILE,), jnp.int32),
        pltpu.VMEM((ETILE,), jnp.int32),
        pltpu.VMEM((WIN, HALF), jnp.float32),
        pltpu.VMEM((WIN, HALF), jnp.float32),
        pltpu.SemaphoreType.DMA,
        pltpu.SemaphoreType.DMA,
        pltpu.SemaphoreType.DMA,
        pltpu.SemaphoreType.DMA,
        pltpu.VMEM((RT,), jnp.float32),
        pltpu.VMEM((RT,), jnp.float32),
        pltpu.VMEM((CH, HALF), jnp.float32),
        pltpu.VMEM((CH, HALF), jnp.float32),
        pltpu.VMEM((CH, HALF), jnp.float32),
        pltpu.VMEM((CH, HALF), jnp.float32),
        pltpu.VMEM((CH, HALF), jnp.float32),
        pltpu.VMEM((CH, HALF), jnp.float32),
        pltpu.VMEM((HALF,), jnp.float32),
        pltpu.VMEM_SHARED((NPAD, HALF), jnp.float32),
    ],
)
def _cheb_kernel(rowi_hbm, coli_hbm, dis_hbm, c_hbm, bias_hbm,
                 out_hbm, htab_hbm, b0_hbm, b1_hbm,
                 coli_v, rowi_v, gbuf0, gbuf1, gsem0, gsem1, ssem0, ssem1,
                 dis_v, dis2_v, zero_v,
                 accv, cv, bpv, bnv, hv, biasv,
                 acc_sh):
    tid = lax.axis_index("s")
    cid = lax.axis_index("c")
    base = tid * RT
    hoff = cid * NPAD

    pltpu.sync_copy(coli_hbm.at[tid], coli_v)
    pltpu.sync_copy(rowi_hbm.at[tid], rowi_v)
    pltpu.sync_copy(dis_hbm.at[cid, pl.ds(base, RT)], dis_v)
    pltpu.sync_copy(bias_hbm.at[cid], biasv)

    @pl.loop(0, ETILE, step=16)
    def _(j):
        coli_v[pl.ds(j, 16)] = coli_v[pl.ds(j, 16)] + hoff

    @pl.loop(0, RT, step=16)
    def _(j):
        dis2_v[pl.ds(j, 16)] = dis_v[pl.ds(j, 16)] * 2.0

    @pl.loop(0, CH)
    def _(i):
        zero_v[i, pl.ds(0, 16)] = jnp.zeros((16,), jnp.float32)
        zero_v[i, pl.ds(16, 16)] = jnp.zeros((16,), jnp.float32)

    @pl.loop(0, NCH)
    def _(ch):
        r0 = base + ch * CH
        pltpu.sync_copy(c_hbm.at[cid, K - 1, pl.ds(r0, CH), pl.ds(0, HALF)], cv)

        @plsc.parallel_loop(0, CH, unroll=4)
        def _(i):
            s = _splat(dis_v, ch * CH + i)
            hv[i, pl.ds(0, 16)] = cv[i, pl.ds(0, 16)] * s
            hv[i, pl.ds(16, 16)] = cv[i, pl.ds(16, 16)] * s

        pltpu.sync_copy(cv, b0_hbm.at[cid, pl.ds(r0, CH)])
        pltpu.sync_copy(hv, htab_hbm.at[pl.ds(hoff + r0, CH)])
        pltpu.sync_copy(zero_v, acc_sh.at[pl.ds(r0, CH)])

    plsc.subcore_barrier()

    for j, k in enumerate(range(K - 2, -1, -1)):
        first = j == 0
        final = k == 0
        bbuf = b1_hbm if j % 2 == 0 else b0_hbm

        pltpu.async_copy(
            htab_hbm.at[coli_v.at[pl.ds(0, WIN)]], gbuf0, gsem0)

        @pl.loop(0, NWIN, step=2)
        def _(w):
            pltpu.async_copy(
                htab_hbm.at[coli_v.at[pl.ds((w + 1) * WIN, WIN)]], gbuf1, gsem1)
            pltpu.make_async_copy(
                htab_hbm.at[coli_v.at[pl.ds(0, WIN)]], gbuf0, gsem0).wait()
            pltpu.sync_copy(gbuf0, acc_sh.at[rowi_v.at[pl.ds(w * WIN, WIN)]], add=True)

            @pl.when(w + 2 < NWIN)
            def _():
                pltpu.async_copy(
                    htab_hbm.at[coli_v.at[pl.ds((w + 2) * WIN, WIN)]], gbuf0, gsem0)

            pltpu.make_async_copy(
                htab_hbm.at[coli_v.at[pl.ds(0, WIN)]], gbuf1, gsem1).wait()
            pltpu.sync_copy(gbuf1, acc_sh.at[rowi_v.at[pl.ds((w + 1) * WIN, WIN)]], add=True)

        plsc.subcore_barrier()

        @pl.loop(0, NCH)
        def _(ch):
            r0 = base + ch * CH
            pltpu.sync_copy(acc_sh.at[pl.ds(r0, CH)], accv)
            pltpu.sync_copy(c_hbm.at[cid, k, pl.ds(r0, CH), pl.ds(0, HALF)], cv)
            if not first:
                pltpu.sync_copy(bbuf.at[cid, pl.ds(r0, CH)], bpv)

            @plsc.parallel_loop(0, CH, unroll=4)
            def _(i):
                s2 = _splat(dis_v if final else dis2_v, ch * CH + i)
                if not final:
                    s1 = _splat(dis_v, ch * CH + i)
                for f in (0, 16):
                    t = cv[i, pl.ds(f, 16)] - s2 * accv[i, pl.ds(f, 16)]
                    if not first:
                        t = t - bpv[i, pl.ds(f, 16)]
                    if final:
                        t = t + biasv[pl.ds(f, 16)]
                        t = jnp.where(t > 0.0, t, jnp.exp(t) - 1.0)
                        bnv[i, pl.ds(f, 16)] = t
                    else:
                        bnv[i, pl.ds(f, 16)] = t
                        hv[i, pl.ds(f, 16)] = t * s1

            if final:
                pltpu.sync_copy(
                    bnv, out_hbm.at[pl.ds(r0, CH), pl.ds(cid * HALF, HALF)])
            else:
                pltpu.sync_copy(bnv, bbuf.at[cid, pl.ds(r0, CH)])
                pltpu.sync_copy(hv, htab_hbm.at[pl.ds(hoff + r0, CH)])
                pltpu.sync_copy(zero_v, acc_sh.at[pl.ds(r0, CH)])

        if not final:
            plsc.subcore_barrier()


def kernel(x, edge_index, W, b):
    pad_tgt = N + (jnp.arange(EPAD - E, dtype=jnp.int32) % (NPAD - N))
    row = jnp.concatenate([edge_index[0].astype(jnp.int32), pad_tgt])
    col = jnp.concatenate([edge_index[1].astype(jnp.int32), pad_tgt])
    row = row.reshape(NTILE, ETILE)
    col = col.reshape(NTILE, ETILE)
    x_pad = jnp.pad(x, ((0, NPAD - N), (0, 0)))
    bias = b.reshape(NSC, HALF)

    W2 = W.reshape(K, DIN, NSC, HALF).transpose(1, 2, 0, 3).reshape(DIN, NSC * K * HALF)
    c = _projections(x_pad, W2)
    dis = _deg_kernel(row)
    out, _, _, _ = _cheb_kernel(row, col, dis, c, bias)
    return out[:N]
